# baseline (speedup 1.0000x reference)
import functools
import math

import jax
import jax.numpy as jnp
from jax.experimental import pallas as pl

_HEADS = 16
_FACTOR = 5.0
_EPS = 1e-9
_NEG = -3.0e38


def _qkv_body(x_ref, wq_ref, wk_ref, wv_ref, bq_ref, bk_ref, bv_ref,
              q_ref, k_ref, v_ref):
    x = x_ref[...]
    for w_ref, b_ref, o_ref in ((wq_ref, bq_ref, q_ref),
                                (wk_ref, bk_ref, k_ref),
                                (wv_ref, bv_ref, v_ref)):
        o_ref[...] = jax.lax.dot_general(
            x, w_ref[...], (((1,), (1,)), ((), ())),
            preferred_element_type=jnp.float32) + b_ref[...]


def _m_body(q_ref, k_ref, m_ref, *, scale, dh, tc):
    L = k_ref.shape[0]
    tq = q_ref.shape[0]
    for t in range(q_ref.shape[1] // dh):
        q = q_ref[:, t * dh:(t + 1) * dh]
        mx = jnp.full((tq, 1), _NEG, jnp.float32)
        sm = jnp.zeros((tq, 1), jnp.float32)
        for c in range(L // tc):
            kc = k_ref[c * tc:(c + 1) * tc, t * dh:(t + 1) * dh]
            s = jax.lax.dot_general(q, kc, (((1,), (1,)), ((), ())),
                                    preferred_element_type=jnp.float32)
            mx = jnp.maximum(mx, jnp.max(s, axis=1, keepdims=True))
            sm = sm + jnp.sum(s, axis=1, keepdims=True)
        m_ref[t] = (mx - sm * (1.0 / L)) * scale


def _topk_body(m_ref, idx_ref, *, k, L, kp):
    m0 = m_ref[...]
    h = m0.shape[0]
    iota = jax.lax.broadcasted_iota(jnp.int32, (h, L), 1)
    col = jax.lax.broadcasted_iota(jnp.int32, (h, kp), 1)
    idx0 = jnp.full((h, kp), L, jnp.int32)

    def step(j, carry):
        m, idx = carry
        mx = jnp.max(m, axis=1, keepdims=True)
        cand = jnp.where(m >= mx, iota, L)
        sel = jnp.min(cand, axis=1, keepdims=True)
        idx = jnp.where(col == j, sel, idx)
        m = jnp.where(iota == sel, _NEG, m)
        return m, idx

    _, idx = jax.lax.fori_loop(0, k, step, (m0, idx0))
    idx_ref[...] = idx


def _attn_body(idx_ref, q_ref, k_ref, v_ref, o_ref, *, scale, dh):
    L = q_ref.shape[0]
    for t in range(q_ref.shape[1] // dh):
        idxc = idx_ref[t]
        kp = idxc.shape[0]
        q = q_ref[:, t * dh:(t + 1) * dh]
        k = k_ref[:, t * dh:(t + 1) * dh]
        v = v_ref[:, t * dh:(t + 1) * dh]
        iota = jax.lax.broadcasted_iota(jnp.int32, (kp, L), 1)
        p = (iota == idxc).astype(jnp.float32)
        qs = jnp.dot(p, q, preferred_element_type=jnp.float32)
        s = jax.lax.dot_general(qs, k, (((1,), (1,)), ((), ())),
                                preferred_element_type=jnp.float32) * scale
        smax = jnp.max(s, axis=1, keepdims=True)
        e = jnp.exp(s - smax)
        a = e / jnp.sum(e, axis=1, keepdims=True)
        cs = jnp.dot(a, v, preferred_element_type=jnp.float32)
        vmean = jnp.mean(v, axis=0, keepdims=True)
        ctx = jax.lax.dot_general(p, cs - vmean, (((0,), (0,)), ((), ())),
                                  preferred_element_type=jnp.float32) + vmean
        o_ref[:, t * dh:(t + 1) * dh] = ctx


def _out_body(c_ref, w_ref, b_ref, o_ref):
    o_ref[...] = jax.lax.dot_general(
        c_ref[...], w_ref[...], (((1,), (1,)), ((), ())),
        preferred_element_type=jnp.float32) + b_ref[...]


def kernel(x, Wq, bq, Wk, bk, Wv, bv, Wo, bo):
    B, L, D = x.shape
    H = _HEADS
    dh = D // H
    scale = 1.0 / math.sqrt(dh)
    kk = min(L, max(1, int(_FACTOR * math.log(L + _EPS))))
    KP = 64
    x2 = x.reshape(B * L, D)

    TN = 256
    b2 = lambda b: b.reshape(1, D)
    q2, k2, v2 = pl.pallas_call(
        _qkv_body,
        grid=(D // TN,),
        in_specs=[
            pl.BlockSpec((B * L, D), lambda j: (0, 0)),
            pl.BlockSpec((TN, D), lambda j: (j, 0)),
            pl.BlockSpec((TN, D), lambda j: (j, 0)),
            pl.BlockSpec((TN, D), lambda j: (j, 0)),
            pl.BlockSpec((1, TN), lambda j: (0, j)),
            pl.BlockSpec((1, TN), lambda j: (0, j)),
            pl.BlockSpec((1, TN), lambda j: (0, j)),
        ],
        out_specs=[
            pl.BlockSpec((B * L, TN), lambda j: (0, j)),
            pl.BlockSpec((B * L, TN), lambda j: (0, j)),
            pl.BlockSpec((B * L, TN), lambda j: (0, j)),
        ],
        out_shape=[jax.ShapeDtypeStruct((B * L, D), jnp.float32)] * 3,
    )(x2, Wq, Wk, Wv, b2(bq), b2(bk), b2(bv))

    TQ = 512
    HP = 2
    m3 = pl.pallas_call(
        functools.partial(_m_body, scale=scale, dh=dh, tc=512),
        grid=(H // HP, L // TQ),
        in_specs=[
            pl.BlockSpec((TQ, HP * dh), lambda h, i: (i, h)),
            pl.BlockSpec((B * L, HP * dh), lambda h, i: (0, h)),
        ],
        out_specs=pl.BlockSpec((HP, TQ, 1), lambda h, i: (h, i, 0)),
        out_shape=jax.ShapeDtypeStruct((H, B * L, 1), jnp.float32),
    )(q2, k2)

    idx = pl.pallas_call(
        functools.partial(_topk_body, k=kk, L=L, kp=KP),
        in_specs=[pl.BlockSpec((H, B * L), lambda: (0, 0))],
        out_specs=pl.BlockSpec((H, KP), lambda: (0, 0)),
        out_shape=jax.ShapeDtypeStruct((H, KP), jnp.int32),
    )(m3.reshape(H, B * L))

    ctx = pl.pallas_call(
        functools.partial(_attn_body, scale=scale, dh=dh),
        grid=(H // HP,),
        in_specs=[
            pl.BlockSpec((HP, KP, 1), lambda h: (h, 0, 0)),
            pl.BlockSpec((B * L, HP * dh), lambda h: (0, h)),
            pl.BlockSpec((B * L, HP * dh), lambda h: (0, h)),
            pl.BlockSpec((B * L, HP * dh), lambda h: (0, h)),
        ],
        out_specs=pl.BlockSpec((B * L, HP * dh), lambda h: (0, h)),
        out_shape=jax.ShapeDtypeStruct((B * L, D), jnp.float32),
    )(idx.reshape(H, KP, 1), q2, k2, v2)

    TM = 256
    out = pl.pallas_call(
        _out_body,
        grid=(B * L // TM,),
        in_specs=[
            pl.BlockSpec((TM, D), lambda i: (i, 0)),
            pl.BlockSpec((D, D), lambda i: (0, 0)),
            pl.BlockSpec((1, D), lambda i: (0, 0)),
        ],
        out_specs=pl.BlockSpec((TM, D), lambda i: (i, 0)),
        out_shape=jax.ShapeDtypeStruct((B * L, D), jnp.float32),
    )(ctx, Wo, bo.reshape(1, D))

    return out.reshape(B, L, D)


# speedup vs baseline: 1.2201x; 1.2201x over previous
import functools
import math

import jax
import jax.numpy as jnp
from jax.experimental import pallas as pl

_HEADS = 16
_FACTOR = 5.0
_EPS = 1e-9
_NEG = -3.0e38


def _qkv_body(x_ref, wq_ref, wk_ref, wv_ref, bq_ref, bk_ref, bv_ref,
              q_ref, k_ref, v_ref):
    x = x_ref[...]
    for w_ref, b_ref, o_ref in ((wq_ref, bq_ref, q_ref),
                                (wk_ref, bk_ref, k_ref),
                                (wv_ref, bv_ref, v_ref)):
        o_ref[...] = jax.lax.dot_general(
            x, w_ref[...], (((1,), (1,)), ((), ())),
            preferred_element_type=jnp.float32) + b_ref[...]


def _m_body(q_ref, k_ref, m_ref, *, scale, dh):
    L = k_ref.shape[0]
    for t in range(q_ref.shape[1] // dh):
        q = q_ref[:, t * dh:(t + 1) * dh]
        k = k_ref[:, t * dh:(t + 1) * dh]
        s = jax.lax.dot_general(q, k, (((1,), (1,)), ((), ())),
                                preferred_element_type=jnp.float32)
        mx = jnp.max(s, axis=1, keepdims=True)
        sm = jnp.sum(s, axis=1, keepdims=True)
        m_ref[t] = (mx - sm * (1.0 / L)) * scale


def _topk_body(m_ref, idx_ref, *, k, L, kp):
    m0 = m_ref[...]
    h = m0.shape[0]
    iota = jax.lax.broadcasted_iota(jnp.int32, (h, L), 1)
    col = jax.lax.broadcasted_iota(jnp.int32, (h, kp), 1)
    idx0 = jnp.full((h, kp), L, jnp.int32)

    def step(j, carry):
        m, idx = carry
        mx = jnp.max(m, axis=1, keepdims=True)
        cand = jnp.where(m >= mx, iota, L)
        sel = jnp.min(cand, axis=1, keepdims=True)
        idx = jnp.where(col == j, sel, idx)
        m = jnp.where(iota == sel, _NEG, m)
        return m, idx

    _, idx = jax.lax.fori_loop(0, k, step, (m0, idx0))
    idx_ref[...] = idx


def _attn_body(idx_ref, q_ref, k_ref, v_ref, o_ref, *, scale, dh):
    L = q_ref.shape[0]
    for t in range(q_ref.shape[1] // dh):
        idxc = idx_ref[t]
        kp = idxc.shape[0]
        q = q_ref[:, t * dh:(t + 1) * dh]
        k = k_ref[:, t * dh:(t + 1) * dh]
        v = v_ref[:, t * dh:(t + 1) * dh]
        iota = jax.lax.broadcasted_iota(jnp.int32, (kp, L), 1)
        p = (iota == idxc).astype(jnp.float32)
        qs = jnp.dot(p, q, preferred_element_type=jnp.float32)
        s = jax.lax.dot_general(qs, k, (((1,), (1,)), ((), ())),
                                preferred_element_type=jnp.float32) * scale
        smax = jnp.max(s, axis=1, keepdims=True)
        e = jnp.exp(s - smax)
        a = e / jnp.sum(e, axis=1, keepdims=True)
        cs = jnp.dot(a, v, preferred_element_type=jnp.float32)
        vmean = jnp.mean(v, axis=0, keepdims=True)
        ctx = jax.lax.dot_general(p, cs - vmean, (((0,), (0,)), ((), ())),
                                  preferred_element_type=jnp.float32) + vmean
        o_ref[:, t * dh:(t + 1) * dh] = ctx


def _out_body(c_ref, w_ref, b_ref, o_ref):
    o_ref[...] = jax.lax.dot_general(
        c_ref[...], w_ref[...], (((1,), (1,)), ((), ())),
        preferred_element_type=jnp.float32) + b_ref[...]


def kernel(x, Wq, bq, Wk, bk, Wv, bv, Wo, bo):
    B, L, D = x.shape
    H = _HEADS
    dh = D // H
    scale = 1.0 / math.sqrt(dh)
    kk = min(L, max(1, int(_FACTOR * math.log(L + _EPS))))
    KP = 64
    x2 = x.reshape(B * L, D)

    TN = 256
    b2 = lambda b: b.reshape(1, D)
    q2, k2, v2 = pl.pallas_call(
        _qkv_body,
        grid=(D // TN,),
        in_specs=[
            pl.BlockSpec((B * L, D), lambda j: (0, 0)),
            pl.BlockSpec((TN, D), lambda j: (j, 0)),
            pl.BlockSpec((TN, D), lambda j: (j, 0)),
            pl.BlockSpec((TN, D), lambda j: (j, 0)),
            pl.BlockSpec((1, TN), lambda j: (0, j)),
            pl.BlockSpec((1, TN), lambda j: (0, j)),
            pl.BlockSpec((1, TN), lambda j: (0, j)),
        ],
        out_specs=[
            pl.BlockSpec((B * L, TN), lambda j: (0, j)),
            pl.BlockSpec((B * L, TN), lambda j: (0, j)),
            pl.BlockSpec((B * L, TN), lambda j: (0, j)),
        ],
        out_shape=[jax.ShapeDtypeStruct((B * L, D), jnp.float32)] * 3,
    )(x2, Wq, Wk, Wv, b2(bq), b2(bk), b2(bv))

    TQ = 512
    HP = 2
    m3 = pl.pallas_call(
        functools.partial(_m_body, scale=scale, dh=dh),
        grid=(H // HP, L // TQ),
        in_specs=[
            pl.BlockSpec((TQ, HP * dh), lambda h, i: (i, h)),
            pl.BlockSpec((B * L, HP * dh), lambda h, i: (0, h)),
        ],
        out_specs=pl.BlockSpec((HP, TQ, 1), lambda h, i: (h, i, 0)),
        out_shape=jax.ShapeDtypeStruct((H, B * L, 1), jnp.float32),
    )(q2, k2)

    idx = pl.pallas_call(
        functools.partial(_topk_body, k=kk, L=L, kp=KP),
        in_specs=[pl.BlockSpec((H, B * L), lambda: (0, 0))],
        out_specs=pl.BlockSpec((H, KP), lambda: (0, 0)),
        out_shape=jax.ShapeDtypeStruct((H, KP), jnp.int32),
    )(m3.reshape(H, B * L))

    ctx = pl.pallas_call(
        functools.partial(_attn_body, scale=scale, dh=dh),
        grid=(H // HP,),
        in_specs=[
            pl.BlockSpec((HP, KP, 1), lambda h: (h, 0, 0)),
            pl.BlockSpec((B * L, HP * dh), lambda h: (0, h)),
            pl.BlockSpec((B * L, HP * dh), lambda h: (0, h)),
            pl.BlockSpec((B * L, HP * dh), lambda h: (0, h)),
        ],
        out_specs=pl.BlockSpec((B * L, HP * dh), lambda h: (0, h)),
        out_shape=jax.ShapeDtypeStruct((B * L, D), jnp.float32),
    )(idx.reshape(H, KP, 1), q2, k2, v2)

    TM = 256
    out = pl.pallas_call(
        _out_body,
        grid=(B * L // TM,),
        in_specs=[
            pl.BlockSpec((TM, D), lambda i: (i, 0)),
            pl.BlockSpec((D, D), lambda i: (0, 0)),
            pl.BlockSpec((1, D), lambda i: (0, 0)),
        ],
        out_specs=pl.BlockSpec((TM, D), lambda i: (i, 0)),
        out_shape=jax.ShapeDtypeStruct((B * L, D), jnp.float32),
    )(ctx, Wo, bo.reshape(1, D))

    return out.reshape(B, L, D)


# speedup vs baseline: 1.2215x; 1.0011x over previous
import functools
import math

import jax
import jax.numpy as jnp
from jax.experimental import pallas as pl

_HEADS = 16
_FACTOR = 5.0
_EPS = 1e-9
_NEG = -3.0e38


def _qkv_body(x_ref, wq_ref, wk_ref, wv_ref, bq_ref, bk_ref, bv_ref,
              q_ref, k_ref, v_ref):
    x = x_ref[...]
    for w_ref, b_ref, o_ref in ((wq_ref, bq_ref, q_ref),
                                (wk_ref, bk_ref, k_ref),
                                (wv_ref, bv_ref, v_ref)):
        o_ref[...] = jax.lax.dot_general(
            x, w_ref[...], (((1,), (1,)), ((), ())),
            preferred_element_type=jnp.float32) + b_ref[...]


def _m_body(q_ref, k_ref, m_ref, *, scale, dh):
    L = k_ref.shape[0]
    for t in range(q_ref.shape[1] // dh):
        q = q_ref[:, t * dh:(t + 1) * dh]
        k = k_ref[:, t * dh:(t + 1) * dh]
        s = jax.lax.dot_general(q, k, (((1,), (1,)), ((), ())),
                                preferred_element_type=jnp.float32)
        mx = jnp.max(s, axis=1, keepdims=True)
        sm = jnp.sum(s, axis=1, keepdims=True)
        m_ref[t] = (mx - sm * (1.0 / L)) * scale


def _topk_body(m_ref, idx_ref, *, k, L, kp):
    m0 = m_ref[...]
    h = m0.shape[0]
    iota = jax.lax.broadcasted_iota(jnp.int32, (h, L), 1)
    col = jax.lax.broadcasted_iota(jnp.int32, (h, kp), 1)
    idx0 = jnp.full((h, kp), L, jnp.int32)

    def step(j, carry):
        m, idx = carry
        mx = jnp.max(m, axis=1, keepdims=True)
        cand = jnp.where(m >= mx, iota, L)
        sel = jnp.min(cand, axis=1, keepdims=True)
        idx = jnp.where(col == j, sel, idx)
        m = jnp.where(iota == sel, _NEG, m)
        return m, idx

    _, idx = jax.lax.fori_loop(0, k, step, (m0, idx0))
    idx_ref[...] = idx


def _attn_body(idx_ref, q_ref, k_ref, v_ref, o_ref, *, scale, dh):
    L = q_ref.shape[0]
    for t in range(q_ref.shape[1] // dh):
        idxc = idx_ref[t]
        kp = idxc.shape[0]
        q = q_ref[:, t * dh:(t + 1) * dh]
        k = k_ref[:, t * dh:(t + 1) * dh]
        v = v_ref[:, t * dh:(t + 1) * dh]
        iota = jax.lax.broadcasted_iota(jnp.int32, (kp, L), 1)
        p = (iota == idxc).astype(jnp.float32)
        qs = jnp.dot(p, q, preferred_element_type=jnp.float32)
        s = jax.lax.dot_general(qs, k, (((1,), (1,)), ((), ())),
                                preferred_element_type=jnp.float32) * scale
        smax = jnp.max(s, axis=1, keepdims=True)
        e = jnp.exp(s - smax)
        a = e / jnp.sum(e, axis=1, keepdims=True)
        cs = jnp.dot(a, v, preferred_element_type=jnp.float32)
        vmean = jnp.mean(v, axis=0, keepdims=True)
        ctx = jax.lax.dot_general(p, cs - vmean, (((0,), (0,)), ((), ())),
                                  preferred_element_type=jnp.float32) + vmean
        o_ref[:, t * dh:(t + 1) * dh] = ctx


def _out_body(c_ref, w_ref, b_ref, o_ref):
    o_ref[...] = jax.lax.dot_general(
        c_ref[...], w_ref[...], (((1,), (1,)), ((), ())),
        preferred_element_type=jnp.float32) + b_ref[...]


def kernel(x, Wq, bq, Wk, bk, Wv, bv, Wo, bo):
    B, L, D = x.shape
    H = _HEADS
    dh = D // H
    scale = 1.0 / math.sqrt(dh)
    kk = min(L, max(1, int(_FACTOR * math.log(L + _EPS))))
    KP = 64
    x2 = x.reshape(B * L, D)

    TN = 256
    b2 = lambda b: b.reshape(1, D)
    q2, k2, v2 = pl.pallas_call(
        _qkv_body,
        grid=(D // TN,),
        in_specs=[
            pl.BlockSpec((B * L, D), lambda j: (0, 0)),
            pl.BlockSpec((TN, D), lambda j: (j, 0)),
            pl.BlockSpec((TN, D), lambda j: (j, 0)),
            pl.BlockSpec((TN, D), lambda j: (j, 0)),
            pl.BlockSpec((1, TN), lambda j: (0, j)),
            pl.BlockSpec((1, TN), lambda j: (0, j)),
            pl.BlockSpec((1, TN), lambda j: (0, j)),
        ],
        out_specs=[
            pl.BlockSpec((B * L, TN), lambda j: (0, j)),
            pl.BlockSpec((B * L, TN), lambda j: (0, j)),
            pl.BlockSpec((B * L, TN), lambda j: (0, j)),
        ],
        out_shape=[jax.ShapeDtypeStruct((B * L, D), jnp.float32)] * 3,
    )(x2, Wq, Wk, Wv, b2(bq), b2(bk), b2(bv))

    TQ = 512
    HP = 2
    m3 = pl.pallas_call(
        functools.partial(_m_body, scale=scale, dh=dh),
        grid=(H // HP, L // TQ),
        in_specs=[
            pl.BlockSpec((TQ, HP * dh), lambda h, i: (i, h)),
            pl.BlockSpec((B * L, HP * dh), lambda h, i: (0, h)),
        ],
        out_specs=pl.BlockSpec((HP, TQ, 1), lambda h, i: (h, i, 0)),
        out_shape=jax.ShapeDtypeStruct((H, B * L, 1), jnp.float32),
    )(q2, k2)

    idx = pl.pallas_call(
        functools.partial(_topk_body, k=kk, L=L, kp=KP),
        in_specs=[pl.BlockSpec((H, B * L), lambda: (0, 0))],
        out_specs=pl.BlockSpec((H, KP), lambda: (0, 0)),
        out_shape=jax.ShapeDtypeStruct((H, KP), jnp.int32),
    )(m3.reshape(H, B * L))

    HP4 = 4
    ctx = pl.pallas_call(
        functools.partial(_attn_body, scale=scale, dh=dh),
        grid=(H // HP4,),
        in_specs=[
            pl.BlockSpec((HP4, KP, 1), lambda h: (h, 0, 0)),
            pl.BlockSpec((B * L, HP4 * dh), lambda h: (0, h)),
            pl.BlockSpec((B * L, HP4 * dh), lambda h: (0, h)),
            pl.BlockSpec((B * L, HP4 * dh), lambda h: (0, h)),
        ],
        out_specs=pl.BlockSpec((B * L, HP4 * dh), lambda h: (0, h)),
        out_shape=jax.ShapeDtypeStruct((B * L, D), jnp.float32),
    )(idx.reshape(H, KP, 1), q2, k2, v2)

    TM = 256
    out = pl.pallas_call(
        _out_body,
        grid=(B * L // TM,),
        in_specs=[
            pl.BlockSpec((TM, D), lambda i: (i, 0)),
            pl.BlockSpec((D, D), lambda i: (0, 0)),
            pl.BlockSpec((1, D), lambda i: (0, 0)),
        ],
        out_specs=pl.BlockSpec((TM, D), lambda i: (i, 0)),
        out_shape=jax.ShapeDtypeStruct((B * L, D), jnp.float32),
    )(ctx, Wo, bo.reshape(1, D))

    return out.reshape(B, L, D)
